# Initial kernel scaffold; baseline (speedup 1.0000x reference)
import jax
import jax.numpy as jnp
from jax.experimental import pallas as pl


def kernel(node_features, edge_index, edge_features, atom_emb, chir_emb, edge_emb1, edge_emb2, W1, b1, W2, b2, gamma, beta):
    raise NotImplementedError("write your pallas kernel here")


# speedup vs baseline: 2.0850x; 2.0850x over previous
import functools

import jax
import jax.numpy as jnp
from jax import lax
from jax.experimental import pallas as pl
from jax.experimental.pallas import tpu as pltpu, tpu_sc as plsc

N = 10000
E = 320000
D = 128
NL = 3

NC = 2
NS = 16
NW = NC * NS

K = 80
ECH = E // K
CPT = ECH // NW
NCH = N // K
RPT = N // NS
QR = N // 8
QPT = 125

_mesh = plsc.VectorSubcoreMesh(core_axis_name="c", subcore_axis_name="s",
                               num_cores=NC, num_subcores=NS)


def _worker_id():
    return lax.axis_index("s") * NC + lax.axis_index("c")


@functools.partial(
    pl.kernel,
    out_type=jax.ShapeDtypeStruct((N, D), jnp.float32),
    mesh=_mesh,
    scratch_types=[
        pltpu.VMEM((NCH, K), jnp.int32),
        pltpu.VMEM((NCH, K), jnp.int32),
        pltpu.VMEM((K, D), jnp.float32),
        pltpu.VMEM((K, D), jnp.float32),
        pltpu.SemaphoreType.DMA,
    ],
)
def _sc_x(nf0_h, nf1_h, atom_h, chir_h, x_out,
          nf0_v, nf1_v, rowsa_v, rowsb_v, sem):
    w = _worker_id()

    pltpu.sync_copy(nf0_h, nf0_v)
    pltpu.sync_copy(nf1_h, nf1_v)
    for t in range(4):
        j = w + t * NW

        @pl.when(j < NCH)
        def _():
            pltpu.async_copy(atom_h.at[nf0_v.at[j]], rowsa_v, sem).wait()
            pltpu.async_copy(chir_h.at[nf1_v.at[j]], rowsb_v, sem).wait()

            def addrow(r, carry):
                for cc in range(D // 16):
                    sl = pl.ds(cc * 16, 16)
                    rowsa_v[r, sl] = rowsa_v[r, sl] + rowsb_v[r, sl]
                return carry

            lax.fori_loop(0, K, addrow, 0)
            pltpu.sync_copy(rowsa_v, x_out.at[pl.ds(j * K, K)])


@functools.partial(
    pl.kernel,
    out_type=jax.ShapeDtypeStruct((NW, 2, QR // 2, 128), jnp.float32),
    mesh=_mesh,
    scratch_types=[
        pltpu.VMEM((CPT, K), jnp.int32),
        pltpu.VMEM((CPT, K), jnp.int32),
        pltpu.VMEM((CPT, K), jnp.int32),
        pltpu.VMEM((QR // 2, 128), jnp.float32),
    ],
)
def _sc_cnt(ef0_h, ef1_h, dstp_h, cnt_out, ef0_v, ef1_v, dst_v, buf_v):
    w = _worker_id()
    hr = QR // 2

    pltpu.sync_copy(ef0_h.at[w], ef0_v)
    pltpu.sync_copy(ef1_h.at[w], ef1_v)
    pltpu.sync_copy(dstp_h.at[w], dst_v)
    iota = lax.iota(jnp.int32, 16)
    z16v = jnp.zeros((16,), jnp.float32)

    for p in range(2):
        def zero(r, carry):
            for cc in range(8):
                buf_v[r, pl.ds(cc * 16, 16)] = z16v
            return carry

        lax.fori_loop(0, hr, zero, 0)

        def chunk(j, carry):
            for g in range(K // 16):
                e0v = ef0_v[j, pl.ds(g * 16, 16)]
                e1v = ef1_v[j, pl.ds(g * 16, 16)]
                dv = dst_v[j, pl.ds(g * 16, 16)]
                qv = lax.shift_right_logical(dv, 3) - p * hr
                blkv = lax.bitwise_and(dv, 7) * 16
                for t in range(16):
                    q = qv[t]
                    row = (jnp.where(iota == e0v[t], 1.0, 0.0)
                           + jnp.where(iota == e1v[t] + 6, 1.0, 0.0))

                    @pl.when(jnp.logical_and(q >= 0, q < hr))
                    def _():
                        sl = pl.ds(blkv[t], 16)
                        buf_v[q, sl] = buf_v[q, sl] + row
            return carry

        lax.fori_loop(0, CPT, chunk, 0)
        pltpu.sync_copy(buf_v, cnt_out.at[w, p])


HN = N // 2
HR = HN + 8
EPT2 = E // NS
CPT2 = EPT2 // K
ZR = 320


@functools.partial(
    pl.kernel,
    out_type=jax.ShapeDtypeStruct((NC, HR, D), jnp.float32),
    mesh=_mesh,
    scratch_types=[
        pltpu.VMEM((CPT2, K), jnp.uint32),
        pltpu.VMEM((9, D), jnp.float32),
        pltpu.VMEM((K,), jnp.int32),
        pltpu.VMEM((K,), jnp.int32),
        pltpu.VMEM((K, D), jnp.float32),
        pltpu.VMEM((ZR, D), jnp.float32),
        pltpu.VMEM_SHARED((HR, D), jnp.float32),
        pltpu.SemaphoreType.DMA,
    ],
)
def _sc_agg(h_h, pk_h, tab_h, zslab_h, part_out,
            pk_v, tab_v, src_v, dst_v, rows_v, zbuf_v, agg_sh, sem):
    c = lax.axis_index("c")
    s = lax.axis_index("s")
    base = c * HN

    pltpu.sync_copy(zslab_h, zbuf_v)

    @pl.when(s < 15)
    def _():
        pltpu.sync_copy(zbuf_v, agg_sh.at[pl.ds(s * ZR, ZR)])

    @pl.when(s == 15)
    def _():
        pltpu.sync_copy(zbuf_v.at[pl.ds(0, HR - 15 * ZR)],
                        agg_sh.at[pl.ds(15 * ZR, HR - 15 * ZR)])

    plsc.subcore_barrier()

    pltpu.sync_copy(pk_h.at[s], pk_v)
    pltpu.sync_copy(tab_h, tab_v)

    def chunk(j, carry):
        for g in range(K // 16):
            sl = pl.ds(g * 16, 16)
            v = pk_v[j, sl]
            src_v[sl] = lax.convert_element_type(
                lax.bitwise_and(v, jnp.uint32(16383)), jnp.int32)
            loc = lax.convert_element_type(
                lax.bitwise_and(lax.shift_right_logical(v, jnp.uint32(14)),
                                jnp.uint32(16383)), jnp.int32) - base
            inb = jnp.logical_and(loc >= 0, loc < HN)
            dst_v[sl] = jnp.where(inb, loc, HN)
        pltpu.async_copy(h_h.at[src_v], rows_v, sem).wait()
        for g in range(K // 16):
            cv = lax.convert_element_type(
                lax.shift_right_logical(pk_v[j, pl.ds(g * 16, 16)],
                                        jnp.uint32(28)), jnp.int32)
            for t in range(16):
                r = g * 16 + t
                ct = cv[t]
                for cc in range(D // 16):
                    sl2 = pl.ds(cc * 16, 16)
                    rows_v[r, sl2] = rows_v[r, sl2] + tab_v[ct, sl2]
        pltpu.sync_copy(rows_v, agg_sh.at[dst_v], add=True)
        return carry

    lax.fori_loop(0, CPT2, chunk, 0)
    plsc.subcore_barrier()

    @pl.when(s < 15)
    def _():
        pltpu.sync_copy(agg_sh.at[pl.ds(s * ZR, ZR)], zbuf_v)
        pltpu.sync_copy(zbuf_v, part_out.at[c, pl.ds(s * ZR, ZR)])

    @pl.when(s == 15)
    def _():
        pltpu.sync_copy(agg_sh.at[pl.ds(15 * ZR, HR - 15 * ZR)],
                        zbuf_v.at[pl.ds(0, HR - 15 * ZR)])
        pltpu.sync_copy(zbuf_v.at[pl.ds(0, HR - 15 * ZR)],
                        part_out.at[c, pl.ds(15 * ZR, HR - 15 * ZR)])


_RB = 1000


def _cnt_sum_body(p_ref, o_ref):
    o_ref[...] = jnp.sum(p_ref[...], axis=0)


def _tc_cnt_sum(parts):
    return pl.pallas_call(
        _cnt_sum_body,
        grid=(10,),
        in_specs=[pl.BlockSpec((NW, N // 10, 16), lambda i: (0, i, 0))],
        out_specs=pl.BlockSpec((N // 10, 16), lambda i: (i, 0)),
        out_shape=jax.ShapeDtypeStruct((N, 16), jnp.float32),
    )(parts)


def _mlp_body(part_ref, h_ref, se_ref,
              w1_ref, b1_ref, w2_ref, b2_ref, z_ref, stats_ref):
    i = pl.program_id(0)
    agg = part_ref[...] + (h_ref[...] + se_ref[...])
    z1 = jnp.maximum(
        jnp.dot(agg.astype(jnp.bfloat16), w1_ref[...].astype(jnp.bfloat16),
                preferred_element_type=jnp.float32)
        + b1_ref[...], 0.0)
    z = (jnp.dot(z1.astype(jnp.bfloat16), w2_ref[...].astype(jnp.bfloat16),
                 preferred_element_type=jnp.float32) + b2_ref[...])
    z_ref[...] = z
    s0 = jnp.sum(z, axis=0, keepdims=True)
    c = s0 * (1.0 / _RB)
    zc = z - c
    s1 = jnp.sum(zc * zc, axis=0, keepdims=True)
    s2 = c * c * float(_RB)

    @pl.when(i == 0)
    def _():
        stats_ref[...] = jnp.zeros_like(stats_ref)

    stats_ref[...] += jnp.concatenate([s0, s1, s2], axis=0)


def _tc_mlp(part, h, se, w1, b1, w2, b2):
    return pl.pallas_call(
        _mlp_body,
        grid=(N // _RB,),
        in_specs=[
            pl.BlockSpec((_RB, D), lambda i: (i, 0)),
            pl.BlockSpec((_RB, D), lambda i: (i, 0)),
            pl.BlockSpec((1, D), lambda i: (0, 0)),
            pl.BlockSpec((D, 2 * D), lambda i: (0, 0)),
            pl.BlockSpec((1, 2 * D), lambda i: (0, 0)),
            pl.BlockSpec((2 * D, D), lambda i: (0, 0)),
            pl.BlockSpec((1, D), lambda i: (0, 0)),
        ],
        out_specs=[
            pl.BlockSpec((_RB, D), lambda i: (i, 0)),
            pl.BlockSpec((3, D), lambda i: (0, 0)),
        ],
        out_shape=[
            jax.ShapeDtypeStruct((N, D), jnp.float32),
            jax.ShapeDtypeStruct((3, D), jnp.float32),
        ],
    )(part, h, se, w1, b1, w2, b2)


def _norm_body(z_ref, stats_ref, g_ref, bt_ref, o_ref):
    mean = stats_ref[0:1] * (1.0 / N)
    var = (stats_ref[1:2] * (1.0 / N)
           + (stats_ref[2:3] * (1.0 / N) - mean * mean))
    inv = lax.rsqrt(var + 1e-5)
    o_ref[...] = jnp.maximum(
        (z_ref[...] - mean) * (inv * g_ref[...]) + bt_ref[...], 0.0)


def _tc_norm(z, stats, g, bt):
    return pl.pallas_call(
        _norm_body,
        grid=(N // _RB,),
        in_specs=[
            pl.BlockSpec((_RB, D), lambda i: (i, 0)),
            pl.BlockSpec((3, D), lambda i: (0, 0)),
            pl.BlockSpec((1, D), lambda i: (0, 0)),
            pl.BlockSpec((1, D), lambda i: (0, 0)),
        ],
        out_specs=pl.BlockSpec((_RB, D), lambda i: (i, 0)),
        out_shape=jax.ShapeDtypeStruct((N, D), jnp.float32),
    )(z, stats, g, bt)


def kernel(node_features, edge_index, edge_features, atom_emb, chir_emb,
           edge_emb1, edge_emb2, W1, b1, W2, b2, gamma, beta):
    nf0 = node_features[:, 0].reshape(NCH, K)
    nf1 = node_features[:, 1].reshape(NCH, K)
    cmb = (edge_features[:, 0] * 3 + edge_features[:, 1]).astype(jnp.uint32)
    pk = (edge_index[0].astype(jnp.uint32)
          + edge_index[1].astype(jnp.uint32) * jnp.uint32(16384)
          + cmb * jnp.uint32(1 << 28)).reshape(NS, CPT2, K)
    zslab = jnp.zeros((ZR, D), jnp.float32)

    tabs = (edge_emb1[:, :3, None, :]
            + edge_emb2[:, None, :3, :]).reshape(NL, 9, D)
    selfee = (edge_emb1[:, 4] + edge_emb2[:, 0]).reshape(NL, 1, D)

    x = _sc_x(nf0, nf1, atom_emb, chir_emb)

    b1r = b1.reshape(NL, 1, 2 * D)
    b2r = b2.reshape(NL, 1, D)
    gr = gamma.reshape(NL, 1, D)
    btr = beta.reshape(NL, 1, D)

    def layer(i, h):
        idx = lambda a: lax.dynamic_index_in_dim(a, i, 0, keepdims=False)
        parts = _sc_agg(h, pk, idx(tabs), zslab)
        part = jnp.concatenate([parts[0, :HN], parts[1, :HN]], axis=0)
        z, stats = _tc_mlp(part, h, idx(selfee),
                           idx(W1), idx(b1r), idx(W2), idx(b2r))
        return _tc_norm(z, stats, idx(gr), idx(btr))

    return lax.fori_loop(0, NL, layer, x)
